# baseline (speedup 1.0000x reference)
import jax
import jax.numpy as jnp
from jax.experimental import pallas as pl

E = 800000
K = 64
BE = 6400


def _block_kernel(gu_ref, gi_ref, gut_ref, git_ref, out_ref):
    p = gu_ref[...] * gi_ref[...] + gut_ref[...] * git_ref[...]
    col = jax.lax.broadcasted_iota(jnp.int32, (2 * K, 2), 0) // K
    sel = jax.lax.broadcasted_iota(jnp.int32, (2 * K, 2), 1)
    mask = (col == sel).astype(jnp.float32)
    out_ref[...] = jax.lax.dot_general(
        p, mask, (((1,), (0,)), ((), ())),
        preferred_element_type=jnp.float32,
        precision=jax.lax.Precision.HIGHEST,
    )


def kernel(gu, gi, gut, git):
    e2 = E // 2
    b2 = BE // 2
    grid = (e2 // b2,)
    gu2 = gu.reshape(e2, 2 * K)
    gi2 = gi.reshape(e2, 2 * K)
    gut2 = gut.reshape(e2, 2 * K)
    git2 = git.reshape(e2, 2 * K)
    in_spec = pl.BlockSpec((b2, 2 * K), lambda i: (i, 0))
    out = pl.pallas_call(
        _block_kernel,
        grid=grid,
        in_specs=[in_spec, in_spec, in_spec, in_spec],
        out_specs=pl.BlockSpec((b2, 2), lambda i: (i, 0)),
        out_shape=jax.ShapeDtypeStruct((e2, 2), jnp.float32),
    )(gu2, gi2, gut2, git2)
    return out.reshape(E)


# speedup vs baseline: 1.3675x; 1.3675x over previous
import jax
import jax.numpy as jnp
from jax.experimental import pallas as pl

E = 800000
K = 64
BE = 8192


def _block_kernel(gu_ref, gi_ref, gut_ref, git_ref, out_ref):
    p = gu_ref[...] * gi_ref[...] + gut_ref[...] * git_ref[...]
    out_ref[...] = jnp.sum(p, axis=1)


def kernel(gu, gi, gut, git):
    grid = ((E + BE - 1) // BE,)
    in_spec = pl.BlockSpec((BE, K), lambda i: (i, 0))
    out = pl.pallas_call(
        _block_kernel,
        grid=grid,
        in_specs=[in_spec, in_spec, in_spec, in_spec],
        out_specs=pl.BlockSpec((BE,), lambda i: (i,)),
        out_shape=jax.ShapeDtypeStruct((E,), jnp.float32),
    )(gu, gi, gut, git)
    return out


# speedup vs baseline: 1.3825x; 1.0110x over previous
import jax
import jax.numpy as jnp
from jax.experimental import pallas as pl

E = 800000
K = 64
BE = 8192


def _block_kernel(gu_ref, gi_ref, gut_ref, git_ref, out_ref):
    p = gu_ref[...] * gi_ref[...] + gut_ref[...] * git_ref[...]
    out_ref[...] = jnp.sum(p.T, axis=0)


def kernel(gu, gi, gut, git):
    grid = ((E + BE - 1) // BE,)
    in_spec = pl.BlockSpec((BE, K), lambda i: (i, 0))
    out = pl.pallas_call(
        _block_kernel,
        grid=grid,
        in_specs=[in_spec, in_spec, in_spec, in_spec],
        out_specs=pl.BlockSpec((BE,), lambda i: (i,)),
        out_shape=jax.ShapeDtypeStruct((E,), jnp.float32),
    )(gu, gi, gut, git)
    return out
